# baseline (speedup 1.0000x reference)
import functools

import jax
import jax.numpy as jnp
from jax import lax
from jax.experimental import pallas as pl
from jax.experimental.pallas import tpu as pltpu
from jax.experimental.pallas import tpu_sc as plsc

VOCAB = 1000000
EMBED = 64
NTOKENS = 200
BATCH = 4096

TOTAL = BATCH * NTOKENS
NUM_WORKERS = 32
PER_WORKER = TOTAL // NUM_WORKERS
CHUNK = NTOKENS
NCHUNKS = PER_WORKER // CHUNK

_mesh = plsc.VectorSubcoreMesh(core_axis_name="c", subcore_axis_name="s")


@functools.partial(
    pl.kernel,
    mesh=_mesh,
    out_type=jax.ShapeDtypeStruct((TOTAL, EMBED), jnp.float32),
    scratch_types=[
        pltpu.VMEM((CHUNK,), jnp.int32),
        pltpu.VMEM((CHUNK, EMBED), jnp.float32),
        pltpu.VMEM((CHUNK, EMBED), jnp.float32),
        pltpu.SemaphoreType.DMA,
    ],
    compiler_params=pltpu.CompilerParams(use_tc_tiling_on_sc=False),
)
def _embed_sc(tokens_hbm, table_hbm, pos_hbm, out_hbm, idx_v, rows_v, pos_v, sem):
    wid = lax.axis_index("s") * 2 + lax.axis_index("c")
    base = wid * PER_WORKER

    pltpu.sync_copy(pos_hbm, pos_v)

    def chunk_body(g, carry):
        off = base + g * CHUNK
        pltpu.sync_copy(tokens_hbm.at[pl.ds(off, CHUNK)], idx_v)
        pltpu.async_copy(table_hbm.at[idx_v], rows_v, sem).wait()

        def row_body(r, c2):
            for cc in range(EMBED // 16):
                sl = pl.ds(cc * 16, 16)
                rows_v[r, sl] = rows_v[r, sl] + pos_v[r, sl]
            return c2

        lax.fori_loop(0, CHUNK, row_body, 0)
        pltpu.sync_copy(rows_v, out_hbm.at[pl.ds(off, CHUNK)])
        return carry

    lax.fori_loop(0, NCHUNKS, chunk_body, 0)


def kernel(tokens, input_embedding, position_embedding):
    flat = tokens.reshape(-1).astype(jnp.int32)
    out = _embed_sc(flat, input_embedding, position_embedding)
    return out.reshape(BATCH, NTOKENS, EMBED)


# speedup vs baseline: 1.3039x; 1.3039x over previous
import functools

import jax
import jax.numpy as jnp
from jax import lax
from jax.experimental import pallas as pl
from jax.experimental.pallas import tpu as pltpu
from jax.experimental.pallas import tpu_sc as plsc

VOCAB = 1000000
EMBED = 64
NTOKENS = 200
BATCH = 4096

TOTAL = BATCH * NTOKENS
NUM_WORKERS = 32
PER_WORKER = TOTAL // NUM_WORKERS
CHUNK = NTOKENS
NCHUNKS = PER_WORKER // CHUNK
GROUPS = [16] * 12 + [8]

_mesh = plsc.VectorSubcoreMesh(core_axis_name="c", subcore_axis_name="s")


@functools.partial(
    pl.kernel,
    mesh=_mesh,
    out_type=jax.ShapeDtypeStruct((TOTAL, EMBED), jnp.float32),
    scratch_types=[
        pltpu.VMEM((CHUNK + 8,), jnp.int32),
        pltpu.VMEM((CHUNK, EMBED), jnp.float32),
        pltpu.VMEM((CHUNK, EMBED), jnp.float32),
        pltpu.SemaphoreType.DMA,
    ],
    compiler_params=pltpu.CompilerParams(use_tc_tiling_on_sc=True),
)
def _embed_sc(tokens_hbm, table_hbm, pos_hbm, out_hbm, idx_v, rows_v, pos_v, sem):
    wid = lax.axis_index("s") * 2 + lax.axis_index("c")
    base = wid * PER_WORKER

    pltpu.sync_copy(pos_hbm, pos_v)

    def chunk_body(g, carry):
        off = base + g * CHUNK
        pltpu.sync_copy(tokens_hbm.at[pl.ds(off, CHUNK)], idx_v.at[pl.ds(0, CHUNK)])

        j = 0
        for gsz in GROUPS:
            v = idx_v[pl.ds(j, 16)]
            for i in range(gsz):
                t = v[i]
                pltpu.async_copy(
                    table_hbm.at[pl.ds(t, 1)], rows_v.at[pl.ds(j + i, 1)], sem
                )
            j += gsz
        for j2 in range(CHUNK):
            pltpu.make_async_copy(
                table_hbm.at[pl.ds(0, 1)], rows_v.at[pl.ds(j2, 1)], sem
            ).wait()

        def row_body(r, c2):
            for cc in range(EMBED // 16):
                sl = pl.ds(cc * 16, 16)
                rows_v[r, sl] = rows_v[r, sl] + pos_v[r, sl]
            return c2

        lax.fori_loop(0, CHUNK, row_body, 0)
        pltpu.sync_copy(rows_v, out_hbm.at[pl.ds(off, CHUNK)])
        return carry

    lax.fori_loop(0, NCHUNKS, chunk_body, 0)


def kernel(tokens, input_embedding, position_embedding):
    flat = tokens.reshape(-1).astype(jnp.int32)
    out = _embed_sc(flat, input_embedding, position_embedding)
    return out.reshape(BATCH, NTOKENS, EMBED)


# speedup vs baseline: 1.3930x; 1.0684x over previous
import functools

import jax
import jax.numpy as jnp
from jax import lax
from jax.experimental import pallas as pl
from jax.experimental.pallas import tpu as pltpu
from jax.experimental.pallas import tpu_sc as plsc

VOCAB = 1000000
EMBED = 64
NTOKENS = 200
BATCH = 4096

TOTAL = BATCH * NTOKENS
NUM_WORKERS = 32
PER_WORKER = TOTAL // NUM_WORKERS
CHUNK = NTOKENS
NCHUNKS = PER_WORKER // CHUNK
GROUPS = [16] * 12 + [8]

_mesh = plsc.VectorSubcoreMesh(core_axis_name="c", subcore_axis_name="s")


@functools.partial(
    pl.kernel,
    mesh=_mesh,
    out_type=jax.ShapeDtypeStruct((TOTAL, EMBED), jnp.float32),
    scratch_types=[
        pltpu.VMEM((CHUNK + 8,), jnp.int32),
        pltpu.VMEM((CHUNK + 8,), jnp.int32),
        pltpu.VMEM((CHUNK, EMBED), jnp.float32),
        pltpu.VMEM((CHUNK, EMBED), jnp.float32),
        pltpu.VMEM((CHUNK, EMBED), jnp.float32),
        pltpu.SemaphoreType.DMA,
        pltpu.SemaphoreType.DMA,
        pltpu.SemaphoreType.DMA,
        pltpu.SemaphoreType.DMA,
        pltpu.SemaphoreType.DMA,
        pltpu.SemaphoreType.DMA,
    ],
    compiler_params=pltpu.CompilerParams(use_tc_tiling_on_sc=True),
)
def _embed_sc(tokens_hbm, table_hbm, pos_hbm, out_hbm,
              idx_a, idx_b, rows_a, rows_b, pos_v,
              sem_ia, sem_ib, sem_ra, sem_rb, sem_oa, sem_ob):
    wid = lax.axis_index("s") * 2 + lax.axis_index("c")
    base = wid * PER_WORKER
    last = NCHUNKS - 1

    pltpu.sync_copy(pos_hbm, pos_v)

    def fire_rows(idx_v, rows_v, sem):
        j = 0
        for gsz in GROUPS:
            v = idx_v[pl.ds(j, 16)]
            for i in range(gsz):
                t = v[i]
                pltpu.async_copy(
                    table_hbm.at[pl.ds(t, 1)], rows_v.at[pl.ds(j + i, 1)], sem
                )
            j += gsz

    def drain_rows(rows_v, sem):
        for j in range(CHUNK):
            pltpu.make_async_copy(
                table_hbm.at[pl.ds(0, 1)], rows_v.at[pl.ds(j, 1)], sem
            ).wait()

    def add_rows(rows_v):
        def row_body(r, c2):
            for cc in range(EMBED // 16):
                sl = pl.ds(cc * 16, 16)
                rows_v[r, sl] = rows_v[r, sl] + pos_v[r, sl]
            return c2

        lax.fori_loop(0, CHUNK, row_body, 0)

    def idx_fetch(c, idx_v, sem):
        c = jnp.minimum(c, last)
        pltpu.async_copy(tokens_hbm.at[pl.ds(base + c * CHUNK, CHUNK)],
                         idx_v.at[pl.ds(0, CHUNK)], sem)

    def idx_wait(idx_v, sem):
        pltpu.make_async_copy(tokens_hbm.at[pl.ds(0, CHUNK)],
                              idx_v.at[pl.ds(0, CHUNK)], sem).wait()

    def out_start(rows_v, c, sem):
        pltpu.async_copy(rows_v, out_hbm.at[pl.ds(base + c * CHUNK, CHUNK)], sem)

    def out_wait(rows_v, sem):
        pltpu.make_async_copy(rows_v, out_hbm.at[pl.ds(0, CHUNK)], sem).wait()

    pltpu.sync_copy(tokens_hbm.at[pl.ds(base, CHUNK)], idx_a.at[pl.ds(0, CHUNK)])
    fire_rows(idx_a, rows_a, sem_ra)
    idx_fetch(1, idx_b, sem_ib)

    def pair_body(g, carry):
        ca = 2 * g
        idx_wait(idx_b, sem_ib)

        @pl.when(g > 0)
        def _():
            out_wait(rows_b, sem_ob)

        fire_rows(idx_b, rows_b, sem_rb)
        idx_fetch(ca + 2, idx_a, sem_ia)
        drain_rows(rows_a, sem_ra)
        add_rows(rows_a)
        out_start(rows_a, ca, sem_oa)

        idx_wait(idx_a, sem_ia)
        out_wait(rows_a, sem_oa)
        fire_rows(idx_a, rows_a, sem_ra)
        idx_fetch(ca + 3, idx_b, sem_ib)
        drain_rows(rows_b, sem_rb)
        add_rows(rows_b)
        out_start(rows_b, ca + 1, sem_ob)
        return carry

    lax.fori_loop(0, NCHUNKS // 2, pair_body, 0)

    idx_wait(idx_b, sem_ib)
    drain_rows(rows_a, sem_ra)
    out_wait(rows_b, sem_ob)


def kernel(tokens, input_embedding, position_embedding):
    flat = tokens.reshape(-1).astype(jnp.int32)
    out = _embed_sc(flat, input_embedding, position_embedding)
    return out.reshape(BATCH, NTOKENS, EMBED)
